# Initial kernel scaffold; baseline (speedup 1.0000x reference)
import jax
import jax.numpy as jnp
from jax.experimental import pallas as pl


def kernel(positions, centroids, tables, W1, b1, W2, b2):
    raise NotImplementedError("write your pallas kernel here")


# speedup vs baseline: 3.4612x; 3.4612x over previous
import dataclasses
import functools

import numpy as np
import jax
import jax.numpy as jnp
from jax import lax
from jax.experimental import pallas as pl
from jax.experimental.pallas import tpu as pltpu
from jax.experimental.pallas import tpu_sc as plsc

_E = 8
_L = 5
_F = 2
_T = 131072
_H = 16
_D = _L * _F
_N = 131072
_BASE_RES = 16
_MAX_RES = 128
_GROWTH = float(np.exp((np.log(_MAX_RES) - np.log(_BASE_RES)) / (_L - 1)))
_RES = [int(np.floor(_BASE_RES * _GROWTH ** l)) for l in range(_L)]
_P1 = int(np.array(2654435761, dtype=np.uint32).view(np.int32))
_P2 = int(np.array(805459861, dtype=np.uint32).view(np.int32))

_NC = 2
_NS = 16
_NW = _NC * _NS
_LANES = 16
_PTS = _N // _NW
_C = 256
_NCHUNK = _PTS // _C
_G = _C // _LANES
_NIDX = _L * 8 * _C


def _f32(x):
    return x.astype(jnp.float32)


def _density_sc(px, py, pz, tab, w1, b1v, w2, b2v, cent):
    mesh = plsc.VectorSubcoreMesh(core_axis_name="c", subcore_axis_name="s")
    cp = pltpu.CompilerParams()
    if "needs_layout_passes" in pltpu.CompilerParams.__dataclass_fields__:
        cp = dataclasses.replace(cp, needs_layout_passes=False)
    if "use_tc_tiling_on_sc" in pltpu.CompilerParams.__dataclass_fields__:
        cp = dataclasses.replace(cp, use_tc_tiling_on_sc=False)

    @functools.partial(
        pl.kernel,
        out_type=jax.ShapeDtypeStruct((_N,), jnp.float32),
        mesh=mesh,
        compiler_params=cp,
        scratch_types=[
            pltpu.VMEM((_C,), jnp.float32),
            pltpu.VMEM((_C,), jnp.float32),
            pltpu.VMEM((_C,), jnp.float32),
            pltpu.VMEM((_C,), jnp.int32),
            pltpu.VMEM((_NIDX,), jnp.int32),
            pltpu.VMEM((_NIDX,), jnp.int32),
            pltpu.VMEM((_NIDX, 8), jnp.float32),
            pltpu.VMEM((_C,), jnp.float32),
            pltpu.VMEM((_E * _D * _H,), jnp.float32),
            pltpu.VMEM((_E * _H,), jnp.float32),
            pltpu.VMEM((_E * _H,), jnp.float32),
            pltpu.VMEM((16,), jnp.float32),
            pltpu.VMEM((32,), jnp.float32),
            pltpu.SemaphoreType.DMA,
        ],
    )
    def k(px_h, py_h, pz_h, tab_h, w1_h, b1_h, w2_h, b2_h, cent_h, out_h,
          px_v, py_v, pz_v, e_v, idx_v, sub_v, rows_v, out_v,
          w1_v, b1_v, w2_v, b2_v, cent_v, sem):
        wid = lax.axis_index("s") * _NC + lax.axis_index("c")
        base0 = wid * _PTS

        pltpu.sync_copy(w1_h, w1_v)
        pltpu.sync_copy(b1_h, b1_v)
        pltpu.sync_copy(w2_h, w2_v)
        pltpu.sync_copy(b2_h, b2_v)
        pltpu.sync_copy(cent_h, cent_v)

        lanes = lax.iota(jnp.int32, _LANES)

        @pl.loop(0, _NCHUNK)
        def _chunk(ci):
            base = pl.multiple_of(base0 + ci * _C, _C)
            pltpu.sync_copy(px_h.at[pl.ds(base, _C)], px_v)
            pltpu.sync_copy(py_h.at[pl.ds(base, _C)], py_v)
            pltpu.sync_copy(pz_h.at[pl.ds(base, _C)], pz_v)

            @pl.loop(0, _G)
            def _grp1(g):
                off = pl.multiple_of(g * _LANES, _LANES)
                x = px_v[pl.ds(off, _LANES)]
                y = py_v[pl.ds(off, _LANES)]
                z = pz_v[pl.ds(off, _LANES)]

                best_d = None
                best_e = None
                for e in range(_E):
                    cex = plsc.load_gather(
                        cent_v, [jnp.full((_LANES,), 3 * e, jnp.int32)])
                    cey = plsc.load_gather(
                        cent_v, [jnp.full((_LANES,), 3 * e + 1, jnp.int32)])
                    cez = plsc.load_gather(
                        cent_v, [jnp.full((_LANES,), 3 * e + 2, jnp.int32)])
                    dx = x - cex
                    dy = y - cey
                    dz = z - cez
                    d = dx * dx + dy * dy + dz * dz
                    if e == 0:
                        best_d = d
                        best_e = jnp.zeros((_LANES,), jnp.int32)
                    else:
                        m = d < best_d
                        best_d = jnp.where(m, d, best_d)
                        best_e = jnp.where(m, e, best_e)
                e_v[pl.ds(off, _LANES)] = best_e

                ebase = best_e * (_L * _T // 4)
                for l in range(_L):
                    res = float(_RES[l])
                    xi = (x * res).astype(jnp.int32)
                    yi = (y * res).astype(jnp.int32)
                    zi = (z * res).astype(jnp.int32)
                    yh0 = yi * _P1
                    yh1 = yh0 + _P1
                    zh0 = zi * _P2
                    zh1 = zh0 + _P2
                    xc1 = xi + 1
                    lbase = ebase + l * (_T // 4)
                    for corner in range(8):
                        cx = xc1 if (corner & 1) else xi
                        yh = yh1 if (corner & 2) else yh0
                        zh = zh1 if (corner & 4) else zh0
                        h = (cx ^ yh ^ zh) & (_T - 1)
                        slot = (l * 8 + corner) * _C
                        idx_v[pl.ds(slot + off, _LANES)] = lbase + (
                            jax.lax.shift_right_logical(h, 2))
                        sub_v[pl.ds(slot + off, _LANES)] = (h & 3) * 2

            pltpu.async_copy(tab_h.at[idx_v], rows_v, sem).wait()

            @pl.loop(0, _G)
            def _grp2(g):
                off = pl.multiple_of(g * _LANES, _LANES)
                x = px_v[pl.ds(off, _LANES)]
                y = py_v[pl.ds(off, _LANES)]
                z = pz_v[pl.ds(off, _LANES)]
                ev = e_v[pl.ds(off, _LANES)]

                enc = [jnp.zeros((_LANES,), jnp.float32) for _ in range(_D)]
                for l in range(_L):
                    res = float(_RES[l])
                    xf = x * res
                    yf = y * res
                    zf = z * res
                    xi = xf.astype(jnp.int32)
                    yi = yf.astype(jnp.int32)
                    zi = zf.astype(jnp.int32)
                    fx = xf - _f32(xi)
                    fy = yf - _f32(yi)
                    fz = zf - _f32(zi)
                    wx = (1.0 - fx, fx)
                    wy = (1.0 - fy, fy)
                    wz = (1.0 - fz, fz)
                    for corner in range(8):
                        cw = (wx[corner & 1]
                              * wy[(corner >> 1) & 1]
                              * wz[(corner >> 2) & 1])
                        slot = (l * 8 + corner) * _C
                        ridx = lanes + (slot + off)
                        sub = sub_v[pl.ds(slot + off, _LANES)]
                        f0 = plsc.load_gather(rows_v, [ridx, sub])
                        f1 = plsc.load_gather(rows_v, [ridx, sub + 1])
                        enc[2 * l] = enc[2 * l] + cw * f0
                        enc[2 * l + 1] = enc[2 * l + 1] + cw * f1

                wb = ev * (_D * _H)
                eb = ev * _H
                raw = plsc.load_gather(b2_v, [ev])
                for j in range(_H):
                    hj = plsc.load_gather(b1_v, [eb + j])
                    for i in range(_D):
                        wij = plsc.load_gather(w1_v, [wb + (i * _H + j)])
                        hj = hj + enc[i] * wij
                    hj = jnp.maximum(hj, 0.0)
                    w2j = plsc.load_gather(w2_v, [eb + j])
                    raw = raw + hj * w2j
                out_v[pl.ds(off, _LANES)] = jnp.exp(raw)

            pltpu.sync_copy(out_v, out_h.at[pl.ds(base, _C)])

    return k(px, py, pz, tab, w1, b1v, w2, b2v, cent)


def kernel(positions, centroids, tables, W1, b1, W2, b2):
    pos = positions.reshape(-1, 3)
    posT = pos.T
    px, py, pz = posT[0], posT[1], posT[2]
    tab = tables.reshape(_E * _L * _T * _F // 8, 8)
    w1 = W1.reshape(-1)
    b1v = b1.reshape(-1)
    w2 = W2.reshape(-1)
    b2v = jnp.concatenate([b2.reshape(-1), jnp.zeros((8,), jnp.float32)])
    cent = jnp.concatenate([centroids.reshape(-1), jnp.zeros((8,), jnp.float32)])
    out = _density_sc(px, py, pz, tab, w1, b1v, w2, b2v, cent)
    return out.reshape(positions.shape[:-1] + (1,))


# speedup vs baseline: 48.1872x; 13.9221x over previous
import dataclasses
import functools

import numpy as np
import jax
import jax.numpy as jnp
from jax import lax
from jax.experimental import pallas as pl
from jax.experimental.pallas import tpu as pltpu
from jax.experimental.pallas import tpu_sc as plsc

_E = 8
_L = 5
_F = 2
_T = 131072
_H = 16
_D = _L * _F
_N = 131072
_BASE_RES = 16
_MAX_RES = 128
_GROWTH = float(np.exp((np.log(_MAX_RES) - np.log(_BASE_RES)) / (_L - 1)))
_RES = [int(np.floor(_BASE_RES * _GROWTH ** l)) for l in range(_L)]
_P1 = int(np.array(2654435761, dtype=np.uint32).view(np.int32))
_P2 = int(np.array(805459861, dtype=np.uint32).view(np.int32))

_NC = 2
_NS = 16
_NW = _NC * _NS
_LANES = 16
_PTS = _N // _NW
_C = 64
_NCHUNK = _PTS // _C
_G = _C // _LANES
_NCORN = _L * 8 * _C
_NIDX = 2 * _NCORN
_LROWS = 2 * _T // 8
_FROWS = 16


def _f32(x):
    return x.astype(jnp.float32)


def _density_sc(px, py, pz, tab, w1, b1v, w2, b2v, cent):
    mesh = plsc.VectorSubcoreMesh(core_axis_name="c", subcore_axis_name="s")
    cp = pltpu.CompilerParams()
    if "needs_layout_passes" in pltpu.CompilerParams.__dataclass_fields__:
        cp = dataclasses.replace(cp, needs_layout_passes=False)
    if "use_tc_tiling_on_sc" in pltpu.CompilerParams.__dataclass_fields__:
        cp = dataclasses.replace(cp, use_tc_tiling_on_sc=False)

    buf_types = [
        pltpu.VMEM((_C,), jnp.float32),
        pltpu.VMEM((_C,), jnp.float32),
        pltpu.VMEM((_C,), jnp.float32),
        pltpu.VMEM((_C,), jnp.int32),
        pltpu.VMEM((_NIDX,), jnp.int32),
        pltpu.VMEM((_NCORN,), jnp.int32),
        pltpu.VMEM((_NIDX, 8), jnp.float32),
        pltpu.SemaphoreType.DMA,
    ]

    @functools.partial(
        pl.kernel,
        out_type=jax.ShapeDtypeStruct((_N,), jnp.float32),
        mesh=mesh,
        compiler_params=cp,
        scratch_types=buf_types + buf_types + [
            pltpu.VMEM((_C,), jnp.float32),
            pltpu.VMEM((_E * _D * _H,), jnp.float32),
            pltpu.VMEM((_E * _H,), jnp.float32),
            pltpu.VMEM((_E * _H,), jnp.float32),
            pltpu.VMEM((16,), jnp.float32),
            pltpu.VMEM((32,), jnp.float32),
        ],
    )
    def k(px_h, py_h, pz_h, tab_h, w1_h, b1_h, w2_h, b2_h, cent_h, out_h,
          *refs):
        bufA = refs[0:8]
        bufB = refs[8:16]
        out_v, w1_v, b1_v, w2_v, b2_v, cent_v = refs[16:22]

        wid = lax.axis_index("s") * _NC + lax.axis_index("c")
        base0 = wid * _PTS

        pltpu.sync_copy(w1_h, w1_v)
        pltpu.sync_copy(b1_h, b1_v)
        pltpu.sync_copy(w2_h, w2_v)
        pltpu.sync_copy(b2_h, b2_v)
        pltpu.sync_copy(cent_h, cent_v)

        lanes = lax.iota(jnp.int32, _LANES)

        def pass1(buf, ci):
            px_v, py_v, pz_v, e_v, idx_v, sub_v, _, _ = buf
            base = pl.multiple_of(base0 + ci * _C, _C)
            pltpu.sync_copy(px_h.at[pl.ds(base, _C)], px_v)
            pltpu.sync_copy(py_h.at[pl.ds(base, _C)], py_v)
            pltpu.sync_copy(pz_h.at[pl.ds(base, _C)], pz_v)

            @pl.loop(0, _G)
            def _grp1(g):
                off = pl.multiple_of(g * _LANES, _LANES)
                x = px_v[pl.ds(off, _LANES)]
                y = py_v[pl.ds(off, _LANES)]
                z = pz_v[pl.ds(off, _LANES)]

                best_d = None
                best_e = None
                for e in range(_E):
                    cex = plsc.load_gather(
                        cent_v, [jnp.full((_LANES,), 3 * e, jnp.int32)])
                    cey = plsc.load_gather(
                        cent_v, [jnp.full((_LANES,), 3 * e + 1, jnp.int32)])
                    cez = plsc.load_gather(
                        cent_v, [jnp.full((_LANES,), 3 * e + 2, jnp.int32)])
                    dx = x - cex
                    dy = y - cey
                    dz = z - cez
                    d = dx * dx + dy * dy + dz * dz
                    if e == 0:
                        best_d = d
                        best_e = jnp.zeros((_LANES,), jnp.int32)
                    else:
                        m = d < best_d
                        best_d = jnp.where(m, d, best_d)
                        best_e = jnp.where(m, e, best_e)
                e_v[pl.ds(off, _LANES)] = best_e

                ebase = best_e * (_L * _LROWS)
                for l in range(_L):
                    res = float(_RES[l])
                    xi = (x * res).astype(jnp.int32)
                    yi = (y * res).astype(jnp.int32)
                    zi = (z * res).astype(jnp.int32)
                    yh0 = yi * _P1
                    yh1 = yh0 + _P1
                    zh0 = zi * _P2
                    zh1 = zh0 + _P2
                    xc1 = xi + 1
                    lbase = ebase + l * _LROWS
                    for corner in range(8):
                        cx = xc1 if (corner & 1) else xi
                        yh = yh1 if (corner & 2) else yh0
                        zh = zh1 if (corner & 4) else zh0
                        h = (cx ^ yh ^ zh) & (_T - 1)
                        s3 = jax.lax.shift_right_logical(h, 3)
                        hi = s3 & ~15
                        row0 = lbase + (hi + hi) + (s3 & 15)
                        slot = (l * 8 + corner) * _C
                        idx_v[pl.ds(slot + off, _LANES)] = row0
                        idx_v[pl.ds(_NCORN + slot + off, _LANES)] = (
                            row0 + _FROWS)
                        sub_v[pl.ds(slot + off, _LANES)] = h & 7

        def fire(buf):
            idx_v, rows_v, sem = buf[4], buf[6], buf[7]
            pltpu.async_copy(tab_h.at[idx_v], rows_v, sem)

        def drain(buf):
            idx_v, rows_v, sem = buf[4], buf[6], buf[7]
            pltpu.make_async_copy(tab_h.at[idx_v], rows_v, sem).wait()

        def pass2(buf, ci):
            px_v, py_v, pz_v, e_v, idx_v, sub_v, rows_v, _ = buf
            base = pl.multiple_of(base0 + ci * _C, _C)

            @pl.loop(0, _G)
            def _grp2(g):
                off = pl.multiple_of(g * _LANES, _LANES)
                x = px_v[pl.ds(off, _LANES)]
                y = py_v[pl.ds(off, _LANES)]
                z = pz_v[pl.ds(off, _LANES)]
                ev = e_v[pl.ds(off, _LANES)]

                enc = [jnp.zeros((_LANES,), jnp.float32) for _ in range(_D)]
                for l in range(_L):
                    res = float(_RES[l])
                    xf = x * res
                    yf = y * res
                    zf = z * res
                    xi = xf.astype(jnp.int32)
                    yi = yf.astype(jnp.int32)
                    zi = zf.astype(jnp.int32)
                    fx = xf - _f32(xi)
                    fy = yf - _f32(yi)
                    fz = zf - _f32(zi)
                    wx = (1.0 - fx, fx)
                    wy = (1.0 - fy, fy)
                    wz = (1.0 - fz, fz)
                    for corner in range(8):
                        cw = (wx[corner & 1]
                              * wy[(corner >> 1) & 1]
                              * wz[(corner >> 2) & 1])
                        slot = (l * 8 + corner) * _C
                        ridx = lanes + (slot + off)
                        sub = sub_v[pl.ds(slot + off, _LANES)]
                        f0 = plsc.load_gather(rows_v, [ridx, sub])
                        f1 = plsc.load_gather(rows_v, [ridx + _NCORN, sub])
                        enc[2 * l] = enc[2 * l] + cw * f0
                        enc[2 * l + 1] = enc[2 * l + 1] + cw * f1

                wb = ev * (_D * _H)
                eb = ev * _H
                raw = plsc.load_gather(b2_v, [ev])
                for j in range(_H):
                    hj = plsc.load_gather(b1_v, [eb + j])
                    for i in range(_D):
                        wij = plsc.load_gather(w1_v, [wb + (i * _H + j)])
                        hj = hj + enc[i] * wij
                    hj = jnp.maximum(hj, 0.0)
                    w2j = plsc.load_gather(w2_v, [eb + j])
                    raw = raw + hj * w2j
                out_v[pl.ds(off, _LANES)] = jnp.exp(raw)

            pltpu.sync_copy(out_v, out_h.at[pl.ds(base, _C)])

        pass1(bufA, 0)
        fire(bufA)

        @pl.loop(0, _NCHUNK // 2 - 1)
        def _pair(i):
            c0 = i * 2
            pass1(bufB, c0 + 1)
            fire(bufB)
            drain(bufA)
            pass2(bufA, c0)
            pass1(bufA, c0 + 2)
            fire(bufA)
            drain(bufB)
            pass2(bufB, c0 + 1)

        pass1(bufB, _NCHUNK - 1)
        fire(bufB)
        drain(bufA)
        pass2(bufA, _NCHUNK - 2)
        drain(bufB)
        pass2(bufB, _NCHUNK - 1)

    return k(px, py, pz, tab, w1, b1v, w2, b2v, cent)


def kernel(positions, centroids, tables, W1, b1, W2, b2):
    pos = positions.reshape(-1, 3)
    posT = pos.T
    px, py, pz = posT[0], posT[1], posT[2]
    tab = (tables.reshape(_E, _L, _T // 128, 128, _F)
           .transpose(0, 1, 2, 4, 3)
           .reshape(_E * _L * _T * _F // 8, 8))
    w1 = W1.reshape(-1)
    b1v = b1.reshape(-1)
    w2 = W2.reshape(-1)
    b2v = jnp.concatenate([b2.reshape(-1), jnp.zeros((8,), jnp.float32)])
    cent = jnp.concatenate([centroids.reshape(-1), jnp.zeros((8,), jnp.float32)])
    out = _density_sc(px, py, pz, tab, w1, b1v, w2, b2v, cent)
    return out.reshape(positions.shape[:-1] + (1,))
